# baseline (speedup 1.0000x reference)
import functools

import jax
import jax.numpy as jnp
import numpy as np
from jax import lax
from jax.experimental import pallas as pl
from jax.experimental.pallas import tpu as pltpu
from jax.experimental.pallas import tpu_sc as plsc

N_NODE = 10000
E_EDGE = 320000
D = 128

NUM_CORES = 2
NUM_SUBCORES = 16
CHUNK = 80
EPT = E_EDGE // NUM_SUBCORES
NCHUNK = EPT // CHUNK
NBINS = 64
ROWS_MAIN = 624

DP = D // 2


def _proj_body(xc, xg, wscg, wdcg, wsgc, wdgc, bscg, bdcg, bsgc, bdgc,
               wccg, bccg, wcgc, bcgc, eat, eas,
               hs_cg, hd_cg, hs_gc, hd_gc, gcg, ggc):
    xcv = xc[...]
    xgv = xg[...]
    f32 = jnp.float32

    def pack(v):
        h = lax.bitcast_convert_type(v.astype(jnp.bfloat16), jnp.uint16)
        lo = h[:, :DP].astype(jnp.uint32)
        hi = h[:, DP:].astype(jnp.uint32) << 16
        return lax.bitcast_convert_type(lo | hi, jnp.int32)

    hs_cg[...] = pack(jnp.dot(xcv, wscg[...], preferred_element_type=f32)
                      + bscg[...])
    hd_cg[...] = pack(jnp.dot(xgv, wdcg[...], preferred_element_type=f32)
                      + bdcg[...])
    hs_gc[...] = pack(jnp.dot(xgv, wsgc[...], preferred_element_type=f32)
                      + bsgc[...])
    hd_gc[...] = pack(jnp.dot(xcv, wdgc[...], preferred_element_type=f32)
                      + bdgc[...])
    for wc, bc, gout in ((wccg, bccg, gcg), (wcgc, bcgc, ggc)):
        tp = jnp.dot(eat[...], wc[0:32, :], preferred_element_type=f32)[0:10]
        sp = jnp.dot(eas[...], wc[32:64, :], preferred_element_type=f32)[0:5]
        z = tp[:, None, :] + sp[None, :, :] + bc[...][None, :, :]
        g50 = jax.nn.sigmoid(z.reshape(50, D)) * 0.25
        gout[...] = jnp.concatenate(
            [g50, jnp.zeros((NBINS - 50, D), f32)], axis=0)


def _project(xc, xg, wscg, wdcg, wsgc, wdgc, bscg, bdcg, bsgc, bdgc,
             wccg, bccg, wcgc, bcgc, eat, eas):
    nb = 25
    br = N_NODE // nb
    row = pl.BlockSpec((br, D), lambda i: (i, 0))
    full = lambda s: pl.BlockSpec(s, lambda i: tuple(0 for _ in s))
    return pl.pallas_call(
        _proj_body,
        grid=(nb,),
        in_specs=[row, row] + [full((D, D))] * 4 + [full((1, D))] * 4
        + [full((64, D)), full((1, D)), full((64, D)), full((1, D)),
           full((16, 32)), full((8, 32))],
        out_specs=[pl.BlockSpec((br, DP), lambda i: (i, 0))] * 4
        + [full((NBINS, D))] * 2,
        out_shape=[jax.ShapeDtypeStruct((N_NODE, DP), jnp.int32)] * 4
        + [jax.ShapeDtypeStruct((NBINS, D), jnp.float32)] * 2,
    )(xc, xg, wscg, wdcg, wsgc, wdgc, bscg, bdcg, bsgc, bdgc,
      wccg, bccg, wcgc, bcgc, eat, eas)


def _sc_body(hs0, hd0, gt0, si0, di0, gi0, hs1, hd1, gt1, si1, di1, gi1,
             out0, out1,
             acc, gts,
             sib0, sib1, dib0, dib1, gib0, gib1, sdib0, sdib1,
             hsb0, hsb1, hdb0, hdb1, gb0, gb1,
             sem_i0, sem_i1, sem_h0, sem_h1, sem_d0, sem_d1, sem_g0, sem_g1,
             sem_s0, sem_s1):
    cid = lax.axis_index("c")
    sid = lax.axis_index("s")
    zero16 = jnp.zeros((16,), jnp.float32)

    def zrow(i, carry):
        for j in range(8):
            gb0[i, pl.ds(j * 16, 16)] = zero16
        return carry

    lax.fori_loop(0, CHUNK, zrow, 0)
    rbase = pl.multiple_of(sid * ROWS_MAIN, 8)

    def zfill(rstart, n80, tail):
        for t in range(n80):
            pltpu.sync_copy(gb0.at[:, :],
                            acc.at[pl.ds(rstart + t * CHUNK, CHUNK), :])
        if tail:
            pltpu.sync_copy(gb0.at[pl.ds(0, tail), :],
                            acc.at[pl.ds(rstart + n80 * CHUNK, tail), :])

    @pl.when(sid < 15)
    def _():
        zfill(rbase, 7, 64)

    @pl.when(sid == 15)
    def _():
        zfill(15 * ROWS_MAIN, 8, 0)

    @pl.when(sid == 0)
    def _():
        @pl.when(cid == 0)
        def _():
            pltpu.sync_copy(gt0, gts)

        @pl.when(cid == 1)
        def _():
            pltpu.sync_copy(gt1, gts)

    plsc.subcore_barrier()

    def run_type(hs, hd, si, di, gi, out):
        ebase0 = sid * EPT
        ibufs = ((sib0, dib0, gib0, sem_i0), (sib1, dib1, gib1, sem_i1))
        dbufs = ((hsb0, hdb0, gb0, sem_h0, sem_d0, sem_g0),
                 (hsb1, hdb1, gb1, sem_h1, sem_d1, sem_g1))
        sbufs = ((sdib0, sem_s0), (sdib1, sem_s1))

        def fire_idx(c, p):
            sb, db, gb_, sm = ibufs[p]
            eb = pl.multiple_of(ebase0 + c * CHUNK, 8)
            pltpu.async_copy(si.at[pl.ds(eb, CHUNK)], sb, sm)
            pltpu.async_copy(di.at[pl.ds(eb, CHUNK)], db, sm)
            pltpu.async_copy(gi.at[pl.ds(eb, CHUNK)], gb_, sm)

        def wait_idx(p):
            sb, db, gb_, sm = ibufs[p]
            pltpu.make_async_copy(si.at[pl.ds(0, CHUNK)], sb, sm).wait()
            pltpu.make_async_copy(si.at[pl.ds(0, CHUNK)], db, sm).wait()
            pltpu.make_async_copy(si.at[pl.ds(0, CHUNK)], gb_, sm).wait()

        def fire_rows(p):
            sb, db, gb_, _ = ibufs[p]
            hb, hdb, gb2, sh, sd, sg = dbufs[p]
            pltpu.async_copy(hs.at[sb], hb, sh)
            pltpu.async_copy(hd.at[db], hdb, sd)
            pltpu.async_copy(gts.at[gb_], gb2, sg)

        def wait_rows(p):
            sb, db, gb_, _ = ibufs[p]
            hb, hdb, gb2, sh, sd, sg = dbufs[p]
            pltpu.make_async_copy(hs.at[sb], hb, sh).wait()
            pltpu.make_async_copy(hd.at[db], hdb, sd).wait()
            pltpu.make_async_copy(gts.at[gb_], gb2, sg).wait()

        def fire_scatter(p):
            gb2 = dbufs[p][2]
            sdb, sm = sbufs[p]
            pltpu.async_copy(gb2, acc.at[sdb], sm, add=True)

        def wait_scatter(p):
            gb2 = dbufs[p][2]
            sdb, sm = sbufs[p]
            pltpu.make_async_copy(gb2, acc.at[sdb], sm).wait()

        fire_idx(0, 0)
        fire_idx(1, 1)
        wait_idx(0)
        fire_rows(0)

        himask = jnp.full((16,), -65536, jnp.int32)

        def step(t, carry):
            for p in (0, 1):
                c = 2 * t + p
                wait_rows(p)
                hb, hdb, gb2, _, _, _ = dbufs[p]

                @pl.when(c + 1 < NCHUNK)
                def _():
                    wait_idx(1 - p)

                    @pl.when(c > 0)
                    def _():
                        wait_scatter(1 - p)

                    fire_rows(1 - p)

                def crow(i):
                    for j in range(4):
                        s16 = pl.ds(j * 16, 16)
                        hv = hb[i, s16]
                        dv = hdb[i, s16]
                        hlo = lax.bitcast_convert_type(hv << 16, jnp.float32)
                        dlo = lax.bitcast_convert_type(dv << 16, jnp.float32)
                        hhi = lax.bitcast_convert_type(hv & himask, jnp.float32)
                        dhi = lax.bitcast_convert_type(dv & himask, jnp.float32)
                        sb2 = pl.ds(j * 16 + DP, 16)
                        gb2[i, s16] = (hlo + dlo) * gb2[i, s16]
                        gb2[i, sb2] = (hhi + dhi) * gb2[i, sb2]

                plsc.parallel_loop(0, CHUNK, unroll=4)(crow)

                db = ibufs[p][1]
                sdb = sbufs[p][0]
                for k in range(CHUNK // 16):
                    sdb[pl.ds(k * 16, 16)] = db[pl.ds(k * 16, 16)]
                fire_scatter(p)

                @pl.when(c + 2 < NCHUNK)
                def _():
                    fire_idx(c + 2, p)
            return carry

        lax.fori_loop(0, NCHUNK // 2, step, 0)
        wait_scatter(1)
        plsc.subcore_barrier()

        @pl.when(sid < 15)
        def _():
            pltpu.sync_copy(acc.at[pl.ds(rbase, ROWS_MAIN), :],
                            out.at[pl.ds(rbase, ROWS_MAIN), :])

        @pl.when(sid == 15)
        def _():
            pltpu.sync_copy(acc.at[pl.ds(15 * ROWS_MAIN, 640), :],
                            out.at[pl.ds(15 * ROWS_MAIN, 640), :])

    @pl.when(cid == 0)
    def _():
        run_type(hs0, hd0, si0, di0, gi0, out0)

    @pl.when(cid == 1)
    def _():
        run_type(hs1, hd1, si1, di1, gi1, out1)


def _sc_edge_pass(hs0, hd0, gt0, si0, di0, gi0, hs1, hd1, gt1, si1, di1, gi1):
    mesh = plsc.VectorSubcoreMesh(core_axis_name="c", subcore_axis_name="s",
                                  num_cores=NUM_CORES, num_subcores=NUM_SUBCORES)
    f = pl.kernel(
        _sc_body,
        out_type=(jax.ShapeDtypeStruct((N_NODE, D), jnp.float32),
                  jax.ShapeDtypeStruct((N_NODE, D), jnp.float32)),
        mesh=mesh,
        compiler_params=pltpu.CompilerParams(use_tc_tiling_on_sc=False),
        scratch_types=[
            pltpu.VMEM_SHARED((N_NODE, D), jnp.float32),
            pltpu.VMEM_SHARED((NBINS, D), jnp.float32),
            pltpu.VMEM((CHUNK,), jnp.int32),
            pltpu.VMEM((CHUNK,), jnp.int32),
            pltpu.VMEM((CHUNK,), jnp.int32),
            pltpu.VMEM((CHUNK,), jnp.int32),
            pltpu.VMEM((CHUNK,), jnp.int32),
            pltpu.VMEM((CHUNK,), jnp.int32),
            pltpu.VMEM((CHUNK,), jnp.int32),
            pltpu.VMEM((CHUNK,), jnp.int32),
            pltpu.VMEM((CHUNK, DP), jnp.int32),
            pltpu.VMEM((CHUNK, DP), jnp.int32),
            pltpu.VMEM((CHUNK, DP), jnp.int32),
            pltpu.VMEM((CHUNK, DP), jnp.int32),
            pltpu.VMEM((CHUNK, D), jnp.float32),
            pltpu.VMEM((CHUNK, D), jnp.float32),
            pltpu.SemaphoreType.DMA,
            pltpu.SemaphoreType.DMA,
            pltpu.SemaphoreType.DMA,
            pltpu.SemaphoreType.DMA,
            pltpu.SemaphoreType.DMA,
            pltpu.SemaphoreType.DMA,
            pltpu.SemaphoreType.DMA,
            pltpu.SemaphoreType.DMA,
            pltpu.SemaphoreType.DMA,
            pltpu.SemaphoreType.DMA,
        ],
    )
    return f(hs0, hd0, gt0, si0, di0, gi0, hs1, hd1, gt1, si1, di1, gi1)


def _out_body(ac, ag, wc, bc, wg, bg, oc, og):
    f32 = jnp.float32
    oc[...] = jnp.dot(ac[...], wc[...], preferred_element_type=f32) + bc[...]
    og[...] = jnp.dot(ag[...], wg[...], preferred_element_type=f32) + bg[...]


def _out_proj(ac, ag, wc, bc, wg, bg):
    nb = 10
    br = N_NODE // nb
    row = pl.BlockSpec((br, D), lambda i: (i, 0))
    full = lambda s: pl.BlockSpec(s, lambda i: tuple(0 for _ in s))
    return pl.pallas_call(
        _out_body,
        grid=(nb,),
        in_specs=[row, row, full((D, D)), full((1, D)), full((D, D)), full((1, D))],
        out_specs=[row, row],
        out_shape=[jax.ShapeDtypeStruct((N_NODE, D), jnp.float32)] * 2,
    )(ac, ag, wc, bc, wg, bg)


@jax.jit
def kernel(x_chemical, x_gene, edge_index_cg, edge_index_gc, edge_attr_cg,
           edge_attr_gc, W_src_cg, b_src_cg, W_dst_cg, b_dst_cg, W_cat_cg,
           b_cat_cg, attn_cg, W_src_gc, b_src_gc, W_dst_gc, b_dst_gc, W_cat_gc,
           b_cat_gc, attn_gc, emb_action_type, emb_action_subject,
           W_out_chemical, b_out_chemical, W_out_gene, b_out_gene):
    del attn_cg, attn_gc
    eat = jnp.zeros((16, 32), jnp.float32).at[:10, :].set(emb_action_type)
    eas = jnp.zeros((8, 32), jnp.float32).at[:5, :].set(emb_action_subject)
    r1 = lambda b: b.reshape(1, D)

    hs_cg, hd_cg, hs_gc, hd_gc, gcg, ggc = _project(
        x_chemical, x_gene, W_src_cg, W_dst_cg, W_src_gc, W_dst_gc,
        r1(b_src_cg), r1(b_dst_cg), r1(b_src_gc), r1(b_dst_gc),
        W_cat_cg, r1(b_cat_cg), W_cat_gc, r1(b_cat_gc), eat, eas)

    i32 = jnp.int32
    gi_cg = (edge_attr_cg[:, 0] * 5 + edge_attr_cg[:, 1]).astype(i32)
    gi_gc = (edge_attr_gc[:, 0] * 5 + edge_attr_gc[:, 1]).astype(i32)

    aggr_gene, aggr_chem = _sc_edge_pass(
        hs_cg, hd_cg, gcg,
        edge_index_cg[0].astype(i32), edge_index_cg[1].astype(i32), gi_cg,
        hs_gc, hd_gc, ggc,
        edge_index_gc[0].astype(i32), edge_index_gc[1].astype(i32), gi_gc)

    out_chem, out_gene = _out_proj(
        aggr_chem, aggr_gene, W_out_chemical, r1(b_out_chemical),
        W_out_gene, r1(b_out_gene))
    return (out_chem, out_gene)


# speedup vs baseline: 1.0922x; 1.0922x over previous
import functools

import jax
import jax.numpy as jnp
import numpy as np
from jax import lax
from jax.experimental import pallas as pl
from jax.experimental.pallas import tpu as pltpu
from jax.experimental.pallas import tpu_sc as plsc

N_NODE = 10000
E_EDGE = 320000
D = 128

NUM_CORES = 2
NUM_SUBCORES = 16
CHUNK = 80
EPT = E_EDGE // NUM_SUBCORES
NCHUNK = EPT // CHUNK
NBINS = 64
ROWS_MAIN = 624

DP = D // 2


def _proj_body(xc, xg, wscg, wdcg, wsgc, wdgc, bscg, bdcg, bsgc, bdgc,
               wccg, bccg, wcgc, bcgc, eat, eas,
               hs_cg, hd_cg, hs_gc, hd_gc, gcg, ggc):
    xcv = xc[...]
    xgv = xg[...]
    f32 = jnp.float32

    def pack(v):
        h = lax.bitcast_convert_type(v.astype(jnp.bfloat16), jnp.uint16)
        lo = h[:, :DP].astype(jnp.uint32)
        hi = h[:, DP:].astype(jnp.uint32) << 16
        return lax.bitcast_convert_type(lo | hi, jnp.int32)

    hs_cg[...] = pack(jnp.dot(xcv, wscg[...], preferred_element_type=f32)
                      + bscg[...])
    hd_cg[...] = pack(jnp.dot(xgv, wdcg[...], preferred_element_type=f32)
                      + bdcg[...])
    hs_gc[...] = pack(jnp.dot(xgv, wsgc[...], preferred_element_type=f32)
                      + bsgc[...])
    hd_gc[...] = pack(jnp.dot(xcv, wdgc[...], preferred_element_type=f32)
                      + bdgc[...])
    for wc, bc, gout in ((wccg, bccg, gcg), (wcgc, bcgc, ggc)):
        tp = jnp.dot(eat[...], wc[0:32, :], preferred_element_type=f32)[0:10]
        sp = jnp.dot(eas[...], wc[32:64, :], preferred_element_type=f32)[0:5]
        z = tp[:, None, :] + sp[None, :, :] + bc[...][None, :, :]
        g50 = jax.nn.sigmoid(z.reshape(50, D)) * 0.25
        gout[...] = jnp.concatenate(
            [g50, jnp.zeros((NBINS - 50, D), f32)], axis=0)


def _project(xc, xg, wscg, wdcg, wsgc, wdgc, bscg, bdcg, bsgc, bdgc,
             wccg, bccg, wcgc, bcgc, eat, eas):
    nb = 25
    br = N_NODE // nb
    row = pl.BlockSpec((br, D), lambda i: (i, 0))
    full = lambda s: pl.BlockSpec(s, lambda i: tuple(0 for _ in s))
    return pl.pallas_call(
        _proj_body,
        grid=(nb,),
        in_specs=[row, row] + [full((D, D))] * 4 + [full((1, D))] * 4
        + [full((64, D)), full((1, D)), full((64, D)), full((1, D)),
           full((16, 32)), full((8, 32))],
        out_specs=[pl.BlockSpec((br, DP), lambda i: (i, 0))] * 4
        + [full((NBINS, D))] * 2,
        out_shape=[jax.ShapeDtypeStruct((N_NODE, DP), jnp.int32)] * 4
        + [jax.ShapeDtypeStruct((NBINS, D), jnp.float32)] * 2,
    )(xc, xg, wscg, wdcg, wsgc, wdgc, bscg, bdcg, bsgc, bdgc,
      wccg, bccg, wcgc, bcgc, eat, eas)


def _sc_body(hs0, hd0, gt0, si0, di0, gi0, hs1, hd1, gt1, si1, di1, gi1,
             out0, out1,
             acc, gts,
             sib0, sib1, dib0, dib1, gib0, gib1,
             hsb0, hsb1, hdb0, hdb1, gb0, gb1,
             sem_i0, sem_i1, sem_h0, sem_h1, sem_d0, sem_d1, sem_g0, sem_g1):
    cid = lax.axis_index("c")
    sid = lax.axis_index("s")
    zero16 = jnp.zeros((16,), jnp.float32)

    def zrow(i, carry):
        for j in range(8):
            gb0[i, pl.ds(j * 16, 16)] = zero16
        return carry

    lax.fori_loop(0, CHUNK, zrow, 0)
    rbase = pl.multiple_of(sid * ROWS_MAIN, 8)

    def zfill(rstart, n80, tail):
        for t in range(n80):
            pltpu.sync_copy(gb0.at[:, :],
                            acc.at[pl.ds(rstart + t * CHUNK, CHUNK), :])
        if tail:
            pltpu.sync_copy(gb0.at[pl.ds(0, tail), :],
                            acc.at[pl.ds(rstart + n80 * CHUNK, tail), :])

    @pl.when(sid < 15)
    def _():
        zfill(rbase, 7, 64)

    @pl.when(sid == 15)
    def _():
        zfill(15 * ROWS_MAIN, 8, 0)

    @pl.when(sid == 0)
    def _():
        @pl.when(cid == 0)
        def _():
            pltpu.sync_copy(gt0, gts)

        @pl.when(cid == 1)
        def _():
            pltpu.sync_copy(gt1, gts)

    plsc.subcore_barrier()

    def run_type(hs, hd, si, di, gi, out):
        ebase0 = sid * EPT
        ibufs = ((sib0, dib0, gib0, sem_i0), (sib1, dib1, gib1, sem_i1))
        dbufs = ((hsb0, hdb0, gb0, sem_h0, sem_d0, sem_g0),
                 (hsb1, hdb1, gb1, sem_h1, sem_d1, sem_g1))

        def fire_idx(c, p):
            sb, db, gb_, sm = ibufs[p]
            eb = pl.multiple_of(ebase0 + c * CHUNK, 8)
            pltpu.async_copy(si.at[pl.ds(eb, CHUNK)], sb, sm)
            pltpu.async_copy(di.at[pl.ds(eb, CHUNK)], db, sm)
            pltpu.async_copy(gi.at[pl.ds(eb, CHUNK)], gb_, sm)

        def wait_idx(p):
            sb, db, gb_, sm = ibufs[p]
            pltpu.make_async_copy(si.at[pl.ds(0, CHUNK)], sb, sm).wait()
            pltpu.make_async_copy(si.at[pl.ds(0, CHUNK)], db, sm).wait()
            pltpu.make_async_copy(si.at[pl.ds(0, CHUNK)], gb_, sm).wait()

        def fire_rows(p):
            sb, db, gb_, _ = ibufs[p]
            hb, hdb, gb2, sh, sd, sg = dbufs[p]
            pltpu.async_copy(hs.at[sb], hb, sh)
            pltpu.async_copy(hd.at[db], hdb, sd)
            pltpu.async_copy(gts.at[gb_], gb2, sg)

        def wait_rows(p):
            sb, db, gb_, _ = ibufs[p]
            hb, hdb, gb2, sh, sd, sg = dbufs[p]
            pltpu.make_async_copy(hs.at[sb], hb, sh).wait()
            pltpu.make_async_copy(hd.at[db], hdb, sd).wait()
            pltpu.make_async_copy(gts.at[gb_], gb2, sg).wait()

        fire_idx(0, 0)
        fire_idx(1, 1)
        wait_idx(0)
        fire_rows(0)

        himask = jnp.full((16,), -65536, jnp.int32)

        def step(t, carry):
            for p in (0, 1):
                c = 2 * t + p
                wait_rows(p)
                hb, hdb, gb2, _, _, _ = dbufs[p]

                @pl.when(c + 1 < NCHUNK)
                def _():
                    wait_idx(1 - p)
                    fire_rows(1 - p)

                def crow(i):
                    for j in range(4):
                        s16 = pl.ds(j * 16, 16)
                        hv = hb[i, s16]
                        dv = hdb[i, s16]
                        hlo = lax.bitcast_convert_type(hv << 16, jnp.float32)
                        dlo = lax.bitcast_convert_type(dv << 16, jnp.float32)
                        hhi = lax.bitcast_convert_type(hv & himask, jnp.float32)
                        dhi = lax.bitcast_convert_type(dv & himask, jnp.float32)
                        sb2 = pl.ds(j * 16 + DP, 16)
                        gb2[i, s16] = (hlo + dlo) * gb2[i, s16]
                        gb2[i, sb2] = (hhi + dhi) * gb2[i, sb2]

                plsc.parallel_loop(0, CHUNK, unroll=4)(crow)

                db = ibufs[p][1]
                pltpu.sync_copy(gb2, acc.at[db], add=True)

                @pl.when(c + 2 < NCHUNK)
                def _():
                    fire_idx(c + 2, p)
            return carry

        lax.fori_loop(0, NCHUNK // 2, step, 0)
        plsc.subcore_barrier()

        @pl.when(sid < 15)
        def _():
            pltpu.sync_copy(acc.at[pl.ds(rbase, ROWS_MAIN), :],
                            out.at[pl.ds(rbase, ROWS_MAIN), :])

        @pl.when(sid == 15)
        def _():
            pltpu.sync_copy(acc.at[pl.ds(15 * ROWS_MAIN, 640), :],
                            out.at[pl.ds(15 * ROWS_MAIN, 640), :])

    @pl.when(cid == 0)
    def _():
        run_type(hs0, hd0, si0, di0, gi0, out0)

    @pl.when(cid == 1)
    def _():
        run_type(hs1, hd1, si1, di1, gi1, out1)


def _sc_edge_pass(hs0, hd0, gt0, si0, di0, gi0, hs1, hd1, gt1, si1, di1, gi1):
    mesh = plsc.VectorSubcoreMesh(core_axis_name="c", subcore_axis_name="s",
                                  num_cores=NUM_CORES, num_subcores=NUM_SUBCORES)
    f = pl.kernel(
        _sc_body,
        out_type=(jax.ShapeDtypeStruct((N_NODE, D), jnp.float32),
                  jax.ShapeDtypeStruct((N_NODE, D), jnp.float32)),
        mesh=mesh,
        compiler_params=pltpu.CompilerParams(use_tc_tiling_on_sc=False),
        scratch_types=[
            pltpu.VMEM_SHARED((N_NODE, D), jnp.float32),
            pltpu.VMEM_SHARED((NBINS, D), jnp.float32),
            pltpu.VMEM((CHUNK,), jnp.int32),
            pltpu.VMEM((CHUNK,), jnp.int32),
            pltpu.VMEM((CHUNK,), jnp.int32),
            pltpu.VMEM((CHUNK,), jnp.int32),
            pltpu.VMEM((CHUNK,), jnp.int32),
            pltpu.VMEM((CHUNK,), jnp.int32),
            pltpu.VMEM((CHUNK, DP), jnp.int32),
            pltpu.VMEM((CHUNK, DP), jnp.int32),
            pltpu.VMEM((CHUNK, DP), jnp.int32),
            pltpu.VMEM((CHUNK, DP), jnp.int32),
            pltpu.VMEM((CHUNK, D), jnp.float32),
            pltpu.VMEM((CHUNK, D), jnp.float32),
            pltpu.SemaphoreType.DMA,
            pltpu.SemaphoreType.DMA,
            pltpu.SemaphoreType.DMA,
            pltpu.SemaphoreType.DMA,
            pltpu.SemaphoreType.DMA,
            pltpu.SemaphoreType.DMA,
            pltpu.SemaphoreType.DMA,
            pltpu.SemaphoreType.DMA,
        ],
    )
    return f(hs0, hd0, gt0, si0, di0, gi0, hs1, hd1, gt1, si1, di1, gi1)


def _out_body(ac, ag, wc, bc, wg, bg, oc, og):
    f32 = jnp.float32
    oc[...] = jnp.dot(ac[...], wc[...], preferred_element_type=f32) + bc[...]
    og[...] = jnp.dot(ag[...], wg[...], preferred_element_type=f32) + bg[...]


def _out_proj(ac, ag, wc, bc, wg, bg):
    nb = 10
    br = N_NODE // nb
    row = pl.BlockSpec((br, D), lambda i: (i, 0))
    full = lambda s: pl.BlockSpec(s, lambda i: tuple(0 for _ in s))
    return pl.pallas_call(
        _out_body,
        grid=(nb,),
        in_specs=[row, row, full((D, D)), full((1, D)), full((D, D)), full((1, D))],
        out_specs=[row, row],
        out_shape=[jax.ShapeDtypeStruct((N_NODE, D), jnp.float32)] * 2,
    )(ac, ag, wc, bc, wg, bg)


@jax.jit
def kernel(x_chemical, x_gene, edge_index_cg, edge_index_gc, edge_attr_cg,
           edge_attr_gc, W_src_cg, b_src_cg, W_dst_cg, b_dst_cg, W_cat_cg,
           b_cat_cg, attn_cg, W_src_gc, b_src_gc, W_dst_gc, b_dst_gc, W_cat_gc,
           b_cat_gc, attn_gc, emb_action_type, emb_action_subject,
           W_out_chemical, b_out_chemical, W_out_gene, b_out_gene):
    del attn_cg, attn_gc
    eat = jnp.zeros((16, 32), jnp.float32).at[:10, :].set(emb_action_type)
    eas = jnp.zeros((8, 32), jnp.float32).at[:5, :].set(emb_action_subject)
    r1 = lambda b: b.reshape(1, D)

    hs_cg, hd_cg, hs_gc, hd_gc, gcg, ggc = _project(
        x_chemical, x_gene, W_src_cg, W_dst_cg, W_src_gc, W_dst_gc,
        r1(b_src_cg), r1(b_dst_cg), r1(b_src_gc), r1(b_dst_gc),
        W_cat_cg, r1(b_cat_cg), W_cat_gc, r1(b_cat_gc), eat, eas)

    i32 = jnp.int32
    gi_cg = (edge_attr_cg[:, 0] * 5 + edge_attr_cg[:, 1]).astype(i32)
    gi_gc = (edge_attr_gc[:, 0] * 5 + edge_attr_gc[:, 1]).astype(i32)

    aggr_gene, aggr_chem = _sc_edge_pass(
        hs_cg, hd_cg, gcg,
        edge_index_cg[0].astype(i32), edge_index_cg[1].astype(i32), gi_cg,
        hs_gc, hd_gc, ggc,
        edge_index_gc[0].astype(i32), edge_index_gc[1].astype(i32), gi_gc)

    out_chem, out_gene = _out_proj(
        aggr_chem, aggr_gene, W_out_chemical, r1(b_out_chemical),
        W_out_gene, r1(b_out_gene))
    return (out_chem, out_gene)
